# baseline (speedup 1.0000x reference)
import functools

import jax
import jax.numpy as jnp
from jax import lax
from jax.experimental import pallas as pl
from jax.experimental.pallas import tpu as pltpu
from jax.experimental.pallas import tpu_sc as plsc

_NW = 32
_CHUNK = 128


def _sc_gather(labels, table):
    B = labels.shape[0]
    D = table.shape[1]
    b_per_w = B // _NW
    n_chunks = b_per_w // _CHUNK

    labels3 = labels.reshape(_NW, n_chunks, _CHUNK)
    mesh = plsc.VectorSubcoreMesh(core_axis_name="c", subcore_axis_name="s")

    @functools.partial(
        pl.kernel,
        out_type=jax.ShapeDtypeStruct((B, D), jnp.float32),
        mesh=mesh,
        compiler_params=pltpu.CompilerParams(use_tc_tiling_on_sc=False),
        scratch_types=[
            pltpu.VMEM((n_chunks, _CHUNK), jnp.int32),
            pltpu.VMEM((b_per_w, D), jnp.float32),
            pltpu.SemaphoreType.DMA,
        ],
    )
    def gather_kernel(labels_hbm, table_hbm, out_hbm, idx_v, rows_v, sem):
        wid = lax.axis_index("s") * 2 + lax.axis_index("c")
        base = wid * b_per_w
        pltpu.sync_copy(labels_hbm.at[wid], idx_v)
        copies = [
            pltpu.async_copy(
                table_hbm.at[idx_v.at[j]],
                rows_v.at[pl.ds(j * _CHUNK, _CHUNK)],
                sem,
            )
            for j in range(n_chunks)
        ]
        for c in copies:
            c.wait()
        pltpu.sync_copy(rows_v, out_hbm.at[pl.ds(base, b_per_w)])

    return gather_kernel(labels3, table)


def _ln_head_body(x_ref, g_ref, bt_ref, w_ref, bb_ref, y_ref, u_ref):
    x = x_ref[...]
    mean = jnp.mean(x, axis=-1, keepdims=True)
    xc = x - mean
    var = jnp.mean(xc * xc, axis=-1, keepdims=True)
    inv = lax.rsqrt(var + 1e-5)
    y = xc * inv * g_ref[...] + bt_ref[...]
    y_ref[...] = y
    z = jnp.sum(y * w_ref[...], axis=-1, keepdims=True) + bb_ref[...]
    u_ref[...] = jax.nn.sigmoid(z)


def _tc_ln_head(rows, gamma, beta, W, b):
    B, D = rows.shape
    BLK = 2048
    grid = (B // BLK,)
    g2 = gamma.reshape(1, D)
    bt2 = beta.reshape(1, D)
    w2 = W.reshape(1, D)
    b2 = b.reshape(1, 1)
    return pl.pallas_call(
        _ln_head_body,
        grid=grid,
        in_specs=[
            pl.BlockSpec((BLK, D), lambda i: (i, 0)),
            pl.BlockSpec((1, D), lambda i: (0, 0)),
            pl.BlockSpec((1, D), lambda i: (0, 0)),
            pl.BlockSpec((1, D), lambda i: (0, 0)),
            pl.BlockSpec((1, 1), lambda i: (0, 0)),
        ],
        out_specs=[
            pl.BlockSpec((BLK, D), lambda i: (i, 0)),
            pl.BlockSpec((BLK, 1), lambda i: (i, 0)),
        ],
        out_shape=[
            jax.ShapeDtypeStruct((B, D), jnp.float32),
            jax.ShapeDtypeStruct((B, 1), jnp.float32),
        ],
    )(rows, g2, bt2, w2, b2)


@jax.jit
def kernel(class_labels, emb_table, ln_gamma, ln_beta, unc_W, unc_b):
    rows = _sc_gather(class_labels.astype(jnp.int32), emb_table)
    y, u = _tc_ln_head(rows, ln_gamma, ln_beta, unc_W, unc_b)
    return (y, u)


# speedup vs baseline: 1.6888x; 1.6888x over previous
import functools

import jax
import jax.numpy as jnp
from jax import lax
from jax.experimental import pallas as pl
from jax.experimental.pallas import tpu as pltpu
from jax.experimental.pallas import tpu_sc as plsc

_NW = 32
_CHUNK = 128


def _sc_gather(labels, table):
    B = labels.shape[0]
    D = table.shape[1]
    b_per_w = B // _NW

    mesh = plsc.VectorSubcoreMesh(core_axis_name="c", subcore_axis_name="s")

    @functools.partial(
        pl.kernel,
        out_type=jax.ShapeDtypeStruct((B, D), jnp.float32),
        mesh=mesh,
        scratch_types=[
            pltpu.VMEM((b_per_w,), jnp.int32),
            pltpu.VMEM((b_per_w, D), jnp.float32),
            pltpu.SemaphoreType.DMA,
        ],
    )
    def gather_kernel(labels_hbm, table_hbm, out_hbm, idx_v, rows_v, sem):
        wid = lax.axis_index("s") * 2 + lax.axis_index("c")
        base = wid * b_per_w
        pltpu.sync_copy(labels_hbm.at[pl.ds(base, b_per_w)], idx_v)

        def body(c, carry):
            vec = idx_v[pl.ds(c * 16, 16)]
            for l in range(16):
                i = vec[l]
                pltpu.make_async_copy(
                    table_hbm.at[pl.ds(i, 1)],
                    rows_v.at[pl.ds(c * 16 + l, 1)],
                    sem,
                ).start()
            return carry

        lax.fori_loop(0, b_per_w // 16, body, 0)
        pltpu.make_async_copy(
            table_hbm.at[pl.ds(0, b_per_w)], rows_v, sem
        ).wait()
        pltpu.sync_copy(rows_v, out_hbm.at[pl.ds(base, b_per_w)])

    return gather_kernel(labels, table)


def _ln_head_body(x_ref, g_ref, bt_ref, w_ref, bb_ref, y_ref, u_ref):
    x = x_ref[...]
    mean = jnp.mean(x, axis=-1, keepdims=True)
    xc = x - mean
    var = jnp.mean(xc * xc, axis=-1, keepdims=True)
    inv = lax.rsqrt(var + 1e-5)
    y = xc * inv * g_ref[...] + bt_ref[...]
    y_ref[...] = y
    z = jnp.sum(y * w_ref[...], axis=-1, keepdims=True) + bb_ref[...]
    u_ref[...] = jax.nn.sigmoid(z)


def _tc_ln_head(rows, gamma, beta, W, b):
    B, D = rows.shape
    BLK = 2048
    grid = (B // BLK,)
    g2 = gamma.reshape(1, D)
    bt2 = beta.reshape(1, D)
    w2 = W.reshape(1, D)
    b2 = b.reshape(1, 1)
    return pl.pallas_call(
        _ln_head_body,
        grid=grid,
        in_specs=[
            pl.BlockSpec((BLK, D), lambda i: (i, 0)),
            pl.BlockSpec((1, D), lambda i: (0, 0)),
            pl.BlockSpec((1, D), lambda i: (0, 0)),
            pl.BlockSpec((1, D), lambda i: (0, 0)),
            pl.BlockSpec((1, 1), lambda i: (0, 0)),
        ],
        out_specs=[
            pl.BlockSpec((BLK, D), lambda i: (i, 0)),
            pl.BlockSpec((BLK, 1), lambda i: (i, 0)),
        ],
        out_shape=[
            jax.ShapeDtypeStruct((B, D), jnp.float32),
            jax.ShapeDtypeStruct((B, 1), jnp.float32),
        ],
    )(rows, g2, bt2, w2, b2)


@jax.jit
def kernel(class_labels, emb_table, ln_gamma, ln_beta, unc_W, unc_b):
    rows = _sc_gather(class_labels.astype(jnp.int32), emb_table)
    y, u = _tc_ln_head(rows, ln_gamma, ln_beta, unc_W, unc_b)
    return (y, u)
